# baseline (speedup 1.0000x reference)
import jax
import jax.numpy as jnp
from jax import lax
from jax.experimental import pallas as pl
from jax.experimental.pallas import tpu as pltpu
from jax.experimental.pallas import tpu_sc as plsc

VOCAB = 1000000
EMB = 32
HID = 128
NCLS = 3
B = 16384
L = 200

NC = 2
NS = 16
NW = NC * NS
BPW = B // NW
IDX_COLS = 100
ROWS_PER_BR = L // IDX_COLS
BR_PER_CHUNK = 8
CHUNK_IDX_ROWS = BR_PER_CHUNK * ROWS_PER_BR
CHUNK_FLAT = BR_PER_CHUNK * L
IDX_ROWS_PER_W = BPW * ROWS_PER_BR
NCHUNKS = BPW // BR_PER_CHUNK
UNROLL = 8
NACC = 4
PROBE_DMA_ONLY = True
PROBE_BF16 = True
EMB_W = 16 if PROBE_BF16 else EMB


def _pool_body(x_hbm, emb_hbm, out_hbm, idx0, idx1, rows0, rows1, acc_v,
               sem0, sem1):
    cid = lax.axis_index("c")
    sid = lax.axis_index("s")
    wid = sid * NC + cid
    row0 = wid * IDX_ROWS_PER_W

    zero16 = jnp.zeros((16,), jnp.float32)

    def fire(c, idxbuf, rowsbuf, sem):
        pltpu.sync_copy(
            x_hbm.at[pl.ds(row0 + c * CHUNK_IDX_ROWS, CHUNK_IDX_ROWS)],
            idxbuf)
        for j in range(CHUNK_IDX_ROWS):
            pltpu.async_copy(emb_hbm.at[idxbuf.at[j]],
                             rowsbuf.at[pl.ds(j * IDX_COLS, IDX_COLS)], sem)

    def drain(idxbuf, rowsbuf, sem):
        for j in range(CHUNK_IDX_ROWS):
            pltpu.make_async_copy(
                emb_hbm.at[idxbuf.at[j]],
                rowsbuf.at[pl.ds(j * IDX_COLS, IDX_COLS)], sem).wait()

    def accumulate(c, rowsbuf):
        for b in range(BR_PER_CHUNK):
            lbase = (c * BR_PER_CHUNK + b) * EMB

            def inner(j, accs, _b=b):
                accs = list(accs)
                rr0 = _b * L + j * UNROLL
                for k in range(UNROLL):
                    a = k % NACC
                    accs[2 * a] = accs[2 * a] + rowsbuf[rr0 + k, pl.ds(0, 16)]
                    accs[2 * a + 1] = (accs[2 * a + 1]
                                       + rowsbuf[rr0 + k, pl.ds(16, 16)])
                return tuple(accs)

            accs = lax.fori_loop(0, L // UNROLL, inner,
                                 (zero16,) * (2 * NACC))
            a0 = accs[0]
            a1 = accs[1]
            for a in range(1, NACC):
                a0 = a0 + accs[2 * a]
                a1 = a1 + accs[2 * a + 1]
            acc_v[pl.ds(lbase, 16)] = a0
            acc_v[pl.ds(lbase + 16, 16)] = a1

    fire(0, idx0, rows0, sem0)

    def outer(c2, carry):
        c = c2 * 2
        fire(c + 1, idx1, rows1, sem1)
        drain(idx0, rows0, sem0)
        if PROBE_DMA_ONLY:
            pass
        else:
            accumulate(c, rows0)

        @pl.when(c2 < NCHUNKS // 2 - 1)
        def _():
            fire(c + 2, idx0, rows0, sem0)

        drain(idx1, rows1, sem1)
        if PROBE_DMA_ONLY:
            pass
        else:
            accumulate(c + 1, rows1)
        return carry

    lax.fori_loop(0, NCHUNKS // 2, outer, 0)
    pltpu.sync_copy(acc_v, out_hbm.at[pl.ds(wid * BPW * EMB, BPW * EMB)])


@jax.jit
def _sc_pool(x2d, emb):
    mesh = plsc.VectorSubcoreMesh(core_axis_name="c", subcore_axis_name="s")
    return pl.kernel(
        _pool_body,
        out_type=jax.ShapeDtypeStruct((B * EMB,), jnp.float32),
        mesh=mesh,
        scratch_types=[
            pltpu.VMEM((CHUNK_IDX_ROWS, IDX_COLS), jnp.int32),
            pltpu.VMEM((CHUNK_IDX_ROWS, IDX_COLS), jnp.int32),
            pltpu.VMEM((CHUNK_FLAT, EMB_W),
                       jnp.int32 if PROBE_BF16 else jnp.float32),
            pltpu.VMEM((CHUNK_FLAT, EMB_W),
                       jnp.int32 if PROBE_BF16 else jnp.float32),
            pltpu.VMEM((BPW * EMB,), jnp.float32),
            pltpu.SemaphoreType.DMA,
            pltpu.SemaphoreType.DMA,
        ],
        compiler_params=pltpu.CompilerParams(use_tc_tiling_on_sc=False),
    )(x2d, emb)


def _mlp_body(h_ref, w1_ref, b1_ref, w2_ref, b2_ref, o_ref):
    h = h_ref[...] * jnp.float32(1.0 / L)
    z = jnp.dot(h, w1_ref[...], preferred_element_type=jnp.float32)
    z = jnp.maximum(z + b1_ref[...], 0.0)
    o_ref[...] = jnp.dot(z, w2_ref[...],
                         preferred_element_type=jnp.float32) + b2_ref[...]


def _mlp(pooled, w1t, b1r, w2p, b2p):
    BM = 1024
    grid = (B // BM,)
    return pl.pallas_call(
        _mlp_body,
        grid=grid,
        in_specs=[
            pl.BlockSpec((BM, EMB), lambda i: (i, 0)),
            pl.BlockSpec((EMB, HID), lambda i: (0, 0)),
            pl.BlockSpec((1, HID), lambda i: (0, 0)),
            pl.BlockSpec((HID, 128), lambda i: (0, 0)),
            pl.BlockSpec((1, 128), lambda i: (0, 0)),
        ],
        out_specs=pl.BlockSpec((BM, 128), lambda i: (i, 0)),
        out_shape=jax.ShapeDtypeStruct((B, 128), jnp.float32),
    )(pooled, w1t, b1r, w2p, b2p)


def kernel(x, emb, W1, b1, W2, b2):
    x2d = x.astype(jnp.int32).reshape(B * L // IDX_COLS, IDX_COLS)
    if PROBE_BF16:
        emb_in = jax.lax.bitcast_convert_type(
            emb.astype(jnp.bfloat16).reshape(VOCAB, EMB_W, 2), jnp.int32)
    else:
        emb_in = emb
    pooled = _sc_pool(x2d, emb_in).reshape(B, EMB)
    w1t = W1.T
    w2p = jnp.pad(W2.T, ((0, 0), (0, 128 - NCLS)))
    b2p = jnp.pad(b2, (0, 128 - NCLS)).reshape(1, 128)
    out = _mlp(pooled, w1t, b1.reshape(1, HID), w2p, b2p)
    return out[:, :NCLS]


# speedup vs baseline: 1.6964x; 1.6964x over previous
import jax
import jax.numpy as jnp
from jax import lax
from jax.experimental import pallas as pl
from jax.experimental.pallas import tpu as pltpu
from jax.experimental.pallas import tpu_sc as plsc

VOCAB = 1000000
EMB = 32
HID = 128
NCLS = 3
B = 16384
L = 200

NC = 2
NS = 16
NW = NC * NS
BPW = B // NW
IDX_COLS = 100
ROWS_PER_BR = L // IDX_COLS
BR_PER_CHUNK = 8
CHUNK_IDX_ROWS = BR_PER_CHUNK * ROWS_PER_BR
CHUNK_FLAT = BR_PER_CHUNK * L
IDX_ROWS_PER_W = BPW * ROWS_PER_BR
NCHUNKS = BPW // BR_PER_CHUNK
UNROLL = 8
NACC = 4
PROBE_DMA_ONLY = False
PROBE_BF16 = False
EMB_W = 16 if PROBE_BF16 else EMB


def _pool_body(x_hbm, emb_hbm, out_hbm, idx0, idx1, rows0, rows1, acc_v,
               sem0, sem1):
    cid = lax.axis_index("c")
    sid = lax.axis_index("s")
    wid = sid * NC + cid
    row0 = wid * IDX_ROWS_PER_W

    zero16 = jnp.zeros((16,), jnp.float32)

    def fire(c, idxbuf, rowsbuf, sem):
        pltpu.sync_copy(
            x_hbm.at[pl.ds(row0 + c * CHUNK_IDX_ROWS, CHUNK_IDX_ROWS)],
            idxbuf)
        for j in range(CHUNK_IDX_ROWS):
            pltpu.async_copy(emb_hbm.at[idxbuf.at[j]],
                             rowsbuf.at[pl.ds(j * IDX_COLS, IDX_COLS)], sem)

    def drain(idxbuf, rowsbuf, sem):
        for j in range(CHUNK_IDX_ROWS):
            pltpu.make_async_copy(
                emb_hbm.at[idxbuf.at[j]],
                rowsbuf.at[pl.ds(j * IDX_COLS, IDX_COLS)], sem).wait()

    def accumulate(c, rowsbuf):
        for b in range(BR_PER_CHUNK):
            lbase = (c * BR_PER_CHUNK + b) * EMB

            def inner(j, accs, _b=b):
                accs = list(accs)
                rr0 = _b * L + j * UNROLL
                for k in range(UNROLL):
                    a = k % NACC
                    accs[2 * a] = accs[2 * a] + rowsbuf[rr0 + k, pl.ds(0, 16)]
                    accs[2 * a + 1] = (accs[2 * a + 1]
                                       + rowsbuf[rr0 + k, pl.ds(16, 16)])
                return tuple(accs)

            accs = lax.fori_loop(0, L // UNROLL, inner,
                                 (zero16,) * (2 * NACC))
            a0 = accs[0]
            a1 = accs[1]
            for a in range(1, NACC):
                a0 = a0 + accs[2 * a]
                a1 = a1 + accs[2 * a + 1]
            acc_v[pl.ds(lbase, 16)] = a0
            acc_v[pl.ds(lbase + 16, 16)] = a1

    fire(0, idx0, rows0, sem0)

    def outer(c2, carry):
        c = c2 * 2
        fire(c + 1, idx1, rows1, sem1)
        drain(idx0, rows0, sem0)
        if PROBE_DMA_ONLY:
            pass
        else:
            accumulate(c, rows0)

        @pl.when(c2 < NCHUNKS // 2 - 1)
        def _():
            fire(c + 2, idx0, rows0, sem0)

        drain(idx1, rows1, sem1)
        if PROBE_DMA_ONLY:
            pass
        else:
            accumulate(c + 1, rows1)
        return carry

    lax.fori_loop(0, NCHUNKS // 2, outer, 0)
    pltpu.sync_copy(acc_v, out_hbm.at[pl.ds(wid * BPW * EMB, BPW * EMB)])


@jax.jit
def _sc_pool(x2d, emb):
    mesh = plsc.VectorSubcoreMesh(core_axis_name="c", subcore_axis_name="s")
    return pl.kernel(
        _pool_body,
        out_type=jax.ShapeDtypeStruct((B * EMB,), jnp.float32),
        mesh=mesh,
        scratch_types=[
            pltpu.VMEM((CHUNK_IDX_ROWS, IDX_COLS), jnp.int32),
            pltpu.VMEM((CHUNK_IDX_ROWS, IDX_COLS), jnp.int32),
            pltpu.VMEM((CHUNK_FLAT, EMB_W),
                       jnp.int32 if PROBE_BF16 else jnp.float32),
            pltpu.VMEM((CHUNK_FLAT, EMB_W),
                       jnp.int32 if PROBE_BF16 else jnp.float32),
            pltpu.VMEM((BPW * EMB,), jnp.float32),
            pltpu.SemaphoreType.DMA,
            pltpu.SemaphoreType.DMA,
        ],
        compiler_params=pltpu.CompilerParams(use_tc_tiling_on_sc=False),
    )(x2d, emb)


def _mlp_body(h_ref, w1_ref, b1_ref, w2_ref, b2_ref, o_ref):
    h = h_ref[...] * jnp.float32(1.0 / L)
    z = jnp.dot(h, w1_ref[...], preferred_element_type=jnp.float32)
    z = jnp.maximum(z + b1_ref[...], 0.0)
    o_ref[...] = jnp.dot(z, w2_ref[...],
                         preferred_element_type=jnp.float32) + b2_ref[...]


def _mlp(pooled, w1t, b1r, w2p, b2p):
    BM = 1024
    grid = (B // BM,)
    return pl.pallas_call(
        _mlp_body,
        grid=grid,
        in_specs=[
            pl.BlockSpec((BM, EMB), lambda i: (i, 0)),
            pl.BlockSpec((EMB, HID), lambda i: (0, 0)),
            pl.BlockSpec((1, HID), lambda i: (0, 0)),
            pl.BlockSpec((HID, 128), lambda i: (0, 0)),
            pl.BlockSpec((1, 128), lambda i: (0, 0)),
        ],
        out_specs=pl.BlockSpec((BM, 128), lambda i: (i, 0)),
        out_shape=jax.ShapeDtypeStruct((B, 128), jnp.float32),
    )(pooled, w1t, b1r, w2p, b2p)


def kernel(x, emb, W1, b1, W2, b2):
    x2d = x.astype(jnp.int32).reshape(B * L // IDX_COLS, IDX_COLS)
    if PROBE_BF16:
        emb_in = jax.lax.bitcast_convert_type(
            emb.astype(jnp.bfloat16).reshape(VOCAB, EMB_W, 2), jnp.int32)
    else:
        emb_in = emb
    pooled = _sc_pool(x2d, emb_in).reshape(B, EMB)
    w1t = W1.T
    w2p = jnp.pad(W2.T, ((0, 0), (0, 128 - NCLS)))
    b2p = jnp.pad(b2, (0, 128 - NCLS)).reshape(1, 128)
    out = _mlp(pooled, w1t, b1.reshape(1, HID), w2p, b2p)
    return out[:, :NCLS]


# speedup vs baseline: 1.8049x; 1.0639x over previous
import jax
import jax.numpy as jnp
from jax import lax
from jax.experimental import pallas as pl
from jax.experimental.pallas import tpu as pltpu
from jax.experimental.pallas import tpu_sc as plsc

VOCAB = 1000000
EMB = 32
HID = 128
NCLS = 3
B = 16384
L = 200

NC = 2
NS = 16
NW = NC * NS
BPW = B // NW
IDX_COLS = 100
ROWS_PER_BR = L // IDX_COLS
BR_PER_CHUNK = 8
CHUNK_IDX_ROWS = BR_PER_CHUNK * ROWS_PER_BR
CHUNK_FLAT = BR_PER_CHUNK * L
IDX_ROWS_PER_W = BPW * ROWS_PER_BR
NCHUNKS = BPW // BR_PER_CHUNK
UNROLL = 8
NACC = 4
PROBE_DMA_ONLY = False
PROBE_BF16 = False
EMB_W = 16 if PROBE_BF16 else EMB


def _pool_body(x_hbm, emb_hbm, out_hbm, idx0, idx1, rows0, rows1, acc_v,
               sem0, sem1):
    cid = lax.axis_index("c")
    sid = lax.axis_index("s")
    wid = sid * NC + cid
    row0 = wid * IDX_ROWS_PER_W

    zero16 = jnp.zeros((16,), jnp.float32)

    def fire(c, idxbuf, rowsbuf, sem):
        pltpu.sync_copy(
            x_hbm.at[pl.ds(row0 + c * CHUNK_IDX_ROWS, CHUNK_IDX_ROWS)],
            idxbuf)
        for j in range(CHUNK_IDX_ROWS):
            pltpu.async_copy(emb_hbm.at[idxbuf.at[j]],
                             rowsbuf.at[pl.ds(j * IDX_COLS, IDX_COLS)], sem)

    def drain(idxbuf, rowsbuf, sem):
        for j in range(CHUNK_IDX_ROWS):
            pltpu.make_async_copy(
                emb_hbm.at[idxbuf.at[j]],
                rowsbuf.at[pl.ds(j * IDX_COLS, IDX_COLS)], sem).wait()

    def accumulate(c, rowsbuf):
        for b in range(BR_PER_CHUNK):
            lbase = (c * BR_PER_CHUNK + b) * EMB

            def inner(j, accs, _b=b):
                accs = list(accs)
                rr0 = _b * L + j * UNROLL
                for k in range(UNROLL):
                    a = k % NACC
                    accs[2 * a] = accs[2 * a] + rowsbuf[rr0 + k, pl.ds(0, 16)]
                    accs[2 * a + 1] = (accs[2 * a + 1]
                                       + rowsbuf[rr0 + k, pl.ds(16, 16)])
                return tuple(accs)

            accs = lax.fori_loop(0, L // UNROLL, inner,
                                 (zero16,) * (2 * NACC))
            a0 = accs[0]
            a1 = accs[1]
            for a in range(1, NACC):
                a0 = a0 + accs[2 * a]
                a1 = a1 + accs[2 * a + 1]
            acc_v[pl.ds(lbase, 16)] = a0
            acc_v[pl.ds(lbase + 16, 16)] = a1

    fire(0, idx0, rows0, sem0)

    def outer(c2, carry):
        c = c2 * 2
        fire(c + 1, idx1, rows1, sem1)
        drain(idx0, rows0, sem0)
        if PROBE_DMA_ONLY:
            pass
        else:
            accumulate(c, rows0)

        @pl.when(c2 < NCHUNKS // 2 - 1)
        def _():
            fire(c + 2, idx0, rows0, sem0)

        drain(idx1, rows1, sem1)
        if PROBE_DMA_ONLY:
            pass
        else:
            accumulate(c + 1, rows1)
        return carry

    lax.fori_loop(0, NCHUNKS // 2, outer, 0)
    pltpu.sync_copy(acc_v, out_hbm.at[pl.ds(wid * BPW * EMB, BPW * EMB)])


@jax.jit
def _sc_pool(x2d, emb):
    mesh = plsc.VectorSubcoreMesh(core_axis_name="c", subcore_axis_name="s")
    return pl.kernel(
        _pool_body,
        out_type=jax.ShapeDtypeStruct((B * EMB,), jnp.float32),
        mesh=mesh,
        scratch_types=[
            pltpu.VMEM((CHUNK_IDX_ROWS, IDX_COLS), jnp.int32),
            pltpu.VMEM((CHUNK_IDX_ROWS, IDX_COLS), jnp.int32),
            pltpu.VMEM((CHUNK_FLAT, EMB_W),
                       jnp.int32 if PROBE_BF16 else jnp.float32),
            pltpu.VMEM((CHUNK_FLAT, EMB_W),
                       jnp.int32 if PROBE_BF16 else jnp.float32),
            pltpu.VMEM((BPW * EMB,), jnp.float32),
            pltpu.SemaphoreType.DMA,
            pltpu.SemaphoreType.DMA,
        ],
        compiler_params=pltpu.CompilerParams(use_tc_tiling_on_sc=False),
    )(x2d, emb)


VB = 2048
NVB = (VOCAB + VB - 1) // VB
VOCAB_P = NVB * VB


def _prep_body(et_ref, o_ref):
    q = VB // 4
    parts = [et_ref[:, pl.ds(a * q, q)].T for a in range(4)]
    o_ref[...] = jnp.concatenate(parts, axis=1)


def _prep(embT):
    return pl.pallas_call(
        _prep_body,
        grid=(NVB,),
        in_specs=[pl.BlockSpec((EMB, VB), lambda i: (0, i))],
        out_specs=pl.BlockSpec((VB // 4, 128), lambda i: (i, 0)),
        out_shape=jax.ShapeDtypeStruct((VOCAB_P * EMB // 128, 128),
                                       jnp.float32),
    )(embT)


def _mlp_body(h_ref, w1_ref, b1_ref, w2_ref, b2_ref, o_ref):
    h = h_ref[...] * jnp.float32(1.0 / L)
    z = jnp.dot(h, w1_ref[...], preferred_element_type=jnp.float32)
    z = jnp.maximum(z + b1_ref[...], 0.0)
    o_ref[...] = jnp.dot(z, w2_ref[...],
                         preferred_element_type=jnp.float32) + b2_ref[...]


def _mlp(pooled, w1t, b1r, w2p, b2p):
    BM = 1024
    grid = (B // BM,)
    return pl.pallas_call(
        _mlp_body,
        grid=grid,
        in_specs=[
            pl.BlockSpec((BM, EMB), lambda i: (i, 0)),
            pl.BlockSpec((EMB, HID), lambda i: (0, 0)),
            pl.BlockSpec((1, HID), lambda i: (0, 0)),
            pl.BlockSpec((HID, 128), lambda i: (0, 0)),
            pl.BlockSpec((1, 128), lambda i: (0, 0)),
        ],
        out_specs=pl.BlockSpec((BM, 128), lambda i: (i, 0)),
        out_shape=jax.ShapeDtypeStruct((B, 128), jnp.float32),
    )(pooled, w1t, b1r, w2p, b2p)


def kernel(x, emb, W1, b1, W2, b2):
    xi = x.astype(jnp.int32)
    xp = (xi & -VB) + ((xi & (VB // 4 - 1)) << 2) + ((xi >> 9) & 3)
    x2d = xp.reshape(B * L // IDX_COLS, IDX_COLS)
    emb_in = _prep(emb.T).reshape(VOCAB_P, EMB)
    pooled = _sc_pool(x2d, emb_in).reshape(B, EMB)
    w1t = W1.T
    w2p = jnp.pad(W2.T, ((0, 0), (0, 128 - NCLS)))
    b2p = jnp.pad(b2, (0, 128 - NCLS)).reshape(1, 128)
    out = _mlp(pooled, w1t, b1.reshape(1, HID), w2p, b2p)
    return out[:, :NCLS]


# speedup vs baseline: 2.9061x; 1.6102x over previous
import jax
import jax.numpy as jnp
from jax import lax
from jax.experimental import pallas as pl
from jax.experimental.pallas import tpu as pltpu
from jax.experimental.pallas import tpu_sc as plsc

VOCAB = 1000000
EMB = 32
HID = 128
NCLS = 3
B = 16384
L = 200

NC = 2
NS = 16
NW = NC * NS
BPW = B // NW
IDX_COLS = 100
ROWS_PER_BR = L // IDX_COLS
BR_PER_CHUNK = 8
CHUNK_IDX_ROWS = BR_PER_CHUNK * ROWS_PER_BR
CHUNK_FLAT = BR_PER_CHUNK * L
IDX_ROWS_PER_W = BPW * ROWS_PER_BR
NCHUNKS = BPW // BR_PER_CHUNK
UNROLL = 8
NACC = 4
PROBE_DMA_ONLY = False
PROBE_BF16 = False
EMB_W = 16 if PROBE_BF16 else EMB


def _pool_body(x_hbm, emb_hbm, out_hbm, idx0, idx1, rows0, rows1, acc_v,
               sem0, sem1):
    cid = lax.axis_index("c")
    sid = lax.axis_index("s")
    wid = sid * NC + cid
    row0 = wid * IDX_ROWS_PER_W

    zero16 = jnp.zeros((16,), jnp.float32)

    def fire(c, idxbuf, rowsbuf, sem):
        pltpu.sync_copy(
            x_hbm.at[pl.ds(row0 + c * CHUNK_IDX_ROWS, CHUNK_IDX_ROWS)],
            idxbuf)
        for j in range(CHUNK_IDX_ROWS):
            pltpu.async_copy(emb_hbm.at[idxbuf.at[j]],
                             rowsbuf.at[pl.ds(j * IDX_COLS, IDX_COLS)], sem)

    def drain(idxbuf, rowsbuf, sem):
        for j in range(CHUNK_IDX_ROWS):
            pltpu.make_async_copy(
                emb_hbm.at[idxbuf.at[j]],
                rowsbuf.at[pl.ds(j * IDX_COLS, IDX_COLS)], sem).wait()

    def accumulate(c, rowsbuf):
        for b in range(BR_PER_CHUNK):
            lbase = (c * BR_PER_CHUNK + b) * EMB

            def inner(j, accs, _b=b):
                accs = list(accs)
                rr0 = _b * L + j * UNROLL
                for k in range(UNROLL):
                    a = k % NACC
                    accs[2 * a] = accs[2 * a] + rowsbuf[rr0 + k, pl.ds(0, 16)]
                    accs[2 * a + 1] = (accs[2 * a + 1]
                                       + rowsbuf[rr0 + k, pl.ds(16, 16)])
                return tuple(accs)

            accs = lax.fori_loop(0, L // UNROLL, inner,
                                 (zero16,) * (2 * NACC))
            a0 = accs[0]
            a1 = accs[1]
            for a in range(1, NACC):
                a0 = a0 + accs[2 * a]
                a1 = a1 + accs[2 * a + 1]
            acc_v[pl.ds(lbase, 16)] = a0
            acc_v[pl.ds(lbase + 16, 16)] = a1

    fire(0, idx0, rows0, sem0)

    def outer(c2, carry):
        c = c2 * 2
        fire(c + 1, idx1, rows1, sem1)
        drain(idx0, rows0, sem0)
        if PROBE_DMA_ONLY:
            pass
        else:
            accumulate(c, rows0)

        @pl.when(c2 < NCHUNKS // 2 - 1)
        def _():
            fire(c + 2, idx0, rows0, sem0)

        drain(idx1, rows1, sem1)
        if PROBE_DMA_ONLY:
            pass
        else:
            accumulate(c + 1, rows1)
        return carry

    lax.fori_loop(0, NCHUNKS // 2, outer, 0)
    pltpu.sync_copy(acc_v, out_hbm.at[pl.ds(wid * BPW * EMB, BPW * EMB)])


@jax.jit
def _sc_pool(x2d, emb):
    mesh = plsc.VectorSubcoreMesh(core_axis_name="c", subcore_axis_name="s")
    return pl.kernel(
        _pool_body,
        out_type=jax.ShapeDtypeStruct((B * EMB,), jnp.float32),
        mesh=mesh,
        scratch_types=[
            pltpu.VMEM((CHUNK_IDX_ROWS, IDX_COLS), jnp.int32),
            pltpu.VMEM((CHUNK_IDX_ROWS, IDX_COLS), jnp.int32),
            pltpu.VMEM((CHUNK_FLAT, EMB_W),
                       jnp.int32 if PROBE_BF16 else jnp.float32),
            pltpu.VMEM((CHUNK_FLAT, EMB_W),
                       jnp.int32 if PROBE_BF16 else jnp.float32),
            pltpu.VMEM((BPW * EMB,), jnp.float32),
            pltpu.SemaphoreType.DMA,
            pltpu.SemaphoreType.DMA,
        ],
        compiler_params=pltpu.CompilerParams(use_tc_tiling_on_sc=False),
    )(x2d, emb)


VB = 32768
NVB = (VOCAB + VB - 1) // VB
VOCAB_P = NVB * VB
QSH = (VB // 4).bit_length() - 1


def _prep_body(et_ref, o_ref):
    q = VB // 4
    row = lax.broadcasted_iota(jnp.int32, (EMB, 128), 0)
    col = lax.broadcasted_iota(jnp.int32, (EMB, 128), 1)
    acc = jnp.zeros((q, 128), jnp.float32)
    for a in range(4):
        ea = jnp.where(col == 32 * a + row, 1.0, 0.0).astype(jnp.float32)
        acc = acc + lax.dot_general(
            et_ref[:, pl.ds(a * q, q)], ea, (((0,), (0,)), ((), ())),
            preferred_element_type=jnp.float32)
    o_ref[...] = acc


def _prep(embT):
    return pl.pallas_call(
        _prep_body,
        grid=(NVB,),
        in_specs=[pl.BlockSpec((EMB, VB), lambda i: (0, i))],
        out_specs=pl.BlockSpec((VB // 4, 128), lambda i: (i, 0)),
        out_shape=jax.ShapeDtypeStruct((VOCAB_P * EMB // 128, 128),
                                       jnp.float32),
    )(embT)


def _mlp_body(h_ref, w1_ref, b1_ref, w2_ref, b2_ref, o_ref):
    h = h_ref[...] * jnp.float32(1.0 / L)
    z = jnp.dot(h, w1_ref[...], preferred_element_type=jnp.float32)
    z = jnp.maximum(z + b1_ref[...], 0.0)
    o_ref[...] = jnp.dot(z, w2_ref[...],
                         preferred_element_type=jnp.float32) + b2_ref[...]


def _mlp(pooled, w1t, b1r, w2p, b2p):
    BM = 1024
    grid = (B // BM,)
    return pl.pallas_call(
        _mlp_body,
        grid=grid,
        in_specs=[
            pl.BlockSpec((BM, EMB), lambda i: (i, 0)),
            pl.BlockSpec((EMB, HID), lambda i: (0, 0)),
            pl.BlockSpec((1, HID), lambda i: (0, 0)),
            pl.BlockSpec((HID, 128), lambda i: (0, 0)),
            pl.BlockSpec((1, 128), lambda i: (0, 0)),
        ],
        out_specs=pl.BlockSpec((BM, 128), lambda i: (i, 0)),
        out_shape=jax.ShapeDtypeStruct((B, 128), jnp.float32),
    )(pooled, w1t, b1r, w2p, b2p)


def kernel(x, emb, W1, b1, W2, b2):
    xi = x.astype(jnp.int32)
    xp = (xi & -VB) + ((xi & (VB // 4 - 1)) << 2) + ((xi >> QSH) & 3)
    x2d = xp.reshape(B * L // IDX_COLS, IDX_COLS)
    emb_in = _prep(emb.T).reshape(VOCAB_P, EMB)
    pooled = _sc_pool(x2d, emb_in).reshape(B, EMB)
    w1t = W1.T
    w2p = jnp.pad(W2.T, ((0, 0), (0, 128 - NCLS)))
    b2p = jnp.pad(b2, (0, 128 - NCLS)).reshape(1, 128)
    out = _mlp(pooled, w1t, b1.reshape(1, HID), w2p, b2p)
    return out[:, :NCLS]


# speedup vs baseline: 2.9083x; 1.0008x over previous
import jax
import jax.numpy as jnp
from jax import lax
from jax.experimental import pallas as pl
from jax.experimental.pallas import tpu as pltpu
from jax.experimental.pallas import tpu_sc as plsc

VOCAB = 1000000
EMB = 32
HID = 128
NCLS = 3
B = 16384
L = 200

NC = 2
NS = 16
NW = NC * NS
BPW = B // NW
IDX_COLS = 100
ROWS_PER_BR = L // IDX_COLS
BR_PER_CHUNK = 8
CHUNK_IDX_ROWS = BR_PER_CHUNK * ROWS_PER_BR
CHUNK_FLAT = BR_PER_CHUNK * L
IDX_ROWS_PER_W = BPW * ROWS_PER_BR
NCHUNKS = BPW // BR_PER_CHUNK
UNROLL = 8
NACC = 4


def _pool_body(x_hbm, emb_hbm, out_hbm, idx0, idx1, rows0, rows1, acc_v,
               sem0, sem1):
    cid = lax.axis_index("c")
    sid = lax.axis_index("s")
    wid = sid * NC + cid
    row0 = wid * IDX_ROWS_PER_W

    zero16 = jnp.zeros((16,), jnp.float32)

    def fire(c, idxbuf, rowsbuf, sem):
        pltpu.sync_copy(
            x_hbm.at[pl.ds(row0 + c * CHUNK_IDX_ROWS, CHUNK_IDX_ROWS)],
            idxbuf)
        for j in range(CHUNK_IDX_ROWS):
            pltpu.async_copy(emb_hbm.at[idxbuf.at[j]],
                             rowsbuf.at[pl.ds(j * IDX_COLS, IDX_COLS)], sem)

    def drain(idxbuf, rowsbuf, sem):
        for j in range(CHUNK_IDX_ROWS):
            pltpu.make_async_copy(
                emb_hbm.at[idxbuf.at[j]],
                rowsbuf.at[pl.ds(j * IDX_COLS, IDX_COLS)], sem).wait()

    def accumulate(c, rowsbuf):
        for b in range(BR_PER_CHUNK):
            lbase = (c * BR_PER_CHUNK + b) * EMB

            def inner(j, accs, _b=b):
                accs = list(accs)
                rr0 = _b * L + j * UNROLL
                for k in range(UNROLL):
                    a = k % NACC
                    accs[2 * a] = accs[2 * a] + rowsbuf[rr0 + k, pl.ds(0, 16)]
                    accs[2 * a + 1] = (accs[2 * a + 1]
                                       + rowsbuf[rr0 + k, pl.ds(16, 16)])
                return tuple(accs)

            accs = lax.fori_loop(0, L // UNROLL, inner,
                                 (zero16,) * (2 * NACC))
            a0 = accs[0]
            a1 = accs[1]
            for a in range(1, NACC):
                a0 = a0 + accs[2 * a]
                a1 = a1 + accs[2 * a + 1]
            acc_v[pl.ds(lbase, 16)] = a0
            acc_v[pl.ds(lbase + 16, 16)] = a1

    fire(0, idx0, rows0, sem0)

    def outer(c2, carry):
        c = c2 * 2
        fire(c + 1, idx1, rows1, sem1)
        drain(idx0, rows0, sem0)
        accumulate(c, rows0)

        @pl.when(c2 < NCHUNKS // 2 - 1)
        def _():
            fire(c + 2, idx0, rows0, sem0)

        drain(idx1, rows1, sem1)
        accumulate(c + 1, rows1)
        return carry

    lax.fori_loop(0, NCHUNKS // 2, outer, 0)
    pltpu.sync_copy(acc_v, out_hbm.at[pl.ds(wid * BPW * EMB, BPW * EMB)])


@jax.jit
def _sc_pool(x2d, emb):
    mesh = plsc.VectorSubcoreMesh(core_axis_name="c", subcore_axis_name="s")
    return pl.kernel(
        _pool_body,
        out_type=jax.ShapeDtypeStruct((B * EMB,), jnp.float32),
        mesh=mesh,
        scratch_types=[
            pltpu.VMEM((CHUNK_IDX_ROWS, IDX_COLS), jnp.int32),
            pltpu.VMEM((CHUNK_IDX_ROWS, IDX_COLS), jnp.int32),
            pltpu.VMEM((CHUNK_FLAT, EMB), jnp.float32),
            pltpu.VMEM((CHUNK_FLAT, EMB), jnp.float32),
            pltpu.VMEM((BPW * EMB,), jnp.float32),
            pltpu.SemaphoreType.DMA,
            pltpu.SemaphoreType.DMA,
        ],
        compiler_params=pltpu.CompilerParams(use_tc_tiling_on_sc=False),
    )(x2d, emb)


VB = 32768
NVB = (VOCAB + VB - 1) // VB
VOCAB_P = NVB * VB
QSH = (VB // 4).bit_length() - 1


def _prep_body(et_ref, o_ref):
    q = VB // 4
    row = lax.broadcasted_iota(jnp.int32, (EMB, 128), 0)
    col = lax.broadcasted_iota(jnp.int32, (EMB, 128), 1)
    acc = jnp.zeros((q, 128), jnp.float32)
    for a in range(4):
        ea = jnp.where(col == 32 * a + row, 1.0, 0.0).astype(jnp.float32)
        acc = acc + lax.dot_general(
            et_ref[:, pl.ds(a * q, q)], ea, (((0,), (0,)), ((), ())),
            preferred_element_type=jnp.float32)
    o_ref[...] = acc


def _prep(embT):
    return pl.pallas_call(
        _prep_body,
        grid=(NVB,),
        in_specs=[pl.BlockSpec((EMB, VB), lambda i: (0, i))],
        out_specs=pl.BlockSpec((VB // 4, 128), lambda i: (i, 0)),
        out_shape=jax.ShapeDtypeStruct((VOCAB_P * EMB // 128, 128),
                                       jnp.float32),
    )(embT)


def _mlp_body(h_ref, w1_ref, b1_ref, w2_ref, b2_ref, o_ref):
    h = h_ref[...] * jnp.float32(1.0 / L)
    z = jnp.dot(h, w1_ref[...], preferred_element_type=jnp.float32)
    z = jnp.maximum(z + b1_ref[...], 0.0)
    o_ref[...] = jnp.dot(z, w2_ref[...],
                         preferred_element_type=jnp.float32) + b2_ref[...]


def _mlp(pooled, w1t, b1r, w2p, b2p):
    BM = 1024
    grid = (B // BM,)
    return pl.pallas_call(
        _mlp_body,
        grid=grid,
        in_specs=[
            pl.BlockSpec((BM, EMB), lambda i: (i, 0)),
            pl.BlockSpec((EMB, HID), lambda i: (0, 0)),
            pl.BlockSpec((1, HID), lambda i: (0, 0)),
            pl.BlockSpec((HID, 128), lambda i: (0, 0)),
            pl.BlockSpec((1, 128), lambda i: (0, 0)),
        ],
        out_specs=pl.BlockSpec((BM, 128), lambda i: (i, 0)),
        out_shape=jax.ShapeDtypeStruct((B, 128), jnp.float32),
    )(pooled, w1t, b1r, w2p, b2p)


def kernel(x, emb, W1, b1, W2, b2):
    xi = x.astype(jnp.int32)
    xp = (xi & -VB) + ((xi & (VB // 4 - 1)) << 2) + ((xi >> QSH) & 3)
    x2d = xp.reshape(B * L // IDX_COLS, IDX_COLS)
    emb_in = _prep(emb.T).reshape(VOCAB_P, EMB)
    pooled = _sc_pool(x2d, emb_in).reshape(B, EMB)
    w1t = W1.T
    w2p = jnp.pad(W2.T, ((0, 0), (0, 128 - NCLS)))
    b2p = jnp.pad(b2, (0, 128 - NCLS)).reshape(1, 128)
    out = _mlp(pooled, w1t, b1.reshape(1, HID), w2p, b2p)
    return out[:, :NCLS]


# speedup vs baseline: 3.0819x; 1.0597x over previous
import jax
import jax.numpy as jnp
from jax import lax
from jax.experimental import pallas as pl
from jax.experimental.pallas import tpu as pltpu
from jax.experimental.pallas import tpu_sc as plsc

VOCAB = 1000000
EMB = 32
HID = 128
NCLS = 3
B = 16384
L = 200

NC = 2
NS = 16
NW = NC * NS
BPW = B // NW
IDX_COLS = 100
ROWS_PER_BR = L // IDX_COLS
BR_PER_CHUNK = 8
CHUNK_IDX_ROWS = BR_PER_CHUNK * ROWS_PER_BR
CHUNK_FLAT = BR_PER_CHUNK * L
IDX_ROWS_PER_W = BPW * ROWS_PER_BR
NCHUNKS = BPW // BR_PER_CHUNK
UNROLL = 8
NACC = 4


def _pool_body(x_hbm, emb_hbm, out_hbm, idx0, idx1, rows0, rows1, acc_v,
               sem0, sem1):
    cid = lax.axis_index("c")
    sid = lax.axis_index("s")
    wid = sid * NC + cid
    row0 = wid * BPW

    zero16 = jnp.zeros((16,), jnp.float32)

    def chunk_copies(idxbuf, rowsbuf, sem):
        for j in range(CHUNK_IDX_ROWS):
            br, half = j // 2, j % 2
            off, sz = (0, 104) if half == 0 else (104, 96)
            yield (emb_hbm.at[idxbuf.at[br, pl.ds(off, sz)]],
                   rowsbuf.at[pl.ds(br * L + off, sz)], sem)

    def fire(c, idxbuf, rowsbuf, sem):
        pltpu.sync_copy(
            x_hbm.at[pl.ds(row0 + c * BR_PER_CHUNK, BR_PER_CHUNK)],
            idxbuf)
        for src, dst, s in chunk_copies(idxbuf, rowsbuf, sem):
            pltpu.async_copy(src, dst, s)

    def drain(idxbuf, rowsbuf, sem):
        for src, dst, s in chunk_copies(idxbuf, rowsbuf, sem):
            pltpu.make_async_copy(src, dst, s).wait()

    def accumulate(c, rowsbuf):
        for b in range(BR_PER_CHUNK):
            lbase = (c * BR_PER_CHUNK + b) * EMB

            def inner(j, accs, _b=b):
                accs = list(accs)
                rr0 = _b * L + j * UNROLL
                for k in range(UNROLL):
                    a = k % NACC
                    accs[2 * a] = accs[2 * a] + rowsbuf[rr0 + k, pl.ds(0, 16)]
                    accs[2 * a + 1] = (accs[2 * a + 1]
                                       + rowsbuf[rr0 + k, pl.ds(16, 16)])
                return tuple(accs)

            accs = lax.fori_loop(0, L // UNROLL, inner,
                                 (zero16,) * (2 * NACC))
            a0 = accs[0]
            a1 = accs[1]
            for a in range(1, NACC):
                a0 = a0 + accs[2 * a]
                a1 = a1 + accs[2 * a + 1]
            acc_v[pl.ds(lbase, 16)] = a0
            acc_v[pl.ds(lbase + 16, 16)] = a1

    fire(0, idx0, rows0, sem0)

    def outer(c2, carry):
        c = c2 * 2
        fire(c + 1, idx1, rows1, sem1)
        drain(idx0, rows0, sem0)
        accumulate(c, rows0)

        @pl.when(c2 < NCHUNKS // 2 - 1)
        def _():
            fire(c + 2, idx0, rows0, sem0)

        drain(idx1, rows1, sem1)
        accumulate(c + 1, rows1)
        return carry

    lax.fori_loop(0, NCHUNKS // 2, outer, 0)
    pltpu.sync_copy(acc_v, out_hbm.at[pl.ds(wid * BPW * EMB, BPW * EMB)])


@jax.jit
def _sc_pool(x2d, emb):
    mesh = plsc.VectorSubcoreMesh(core_axis_name="c", subcore_axis_name="s")
    return pl.kernel(
        _pool_body,
        out_type=jax.ShapeDtypeStruct((B * EMB,), jnp.float32),
        mesh=mesh,
        scratch_types=[
            pltpu.VMEM((BR_PER_CHUNK, L), jnp.int32),
            pltpu.VMEM((BR_PER_CHUNK, L), jnp.int32),
            pltpu.VMEM((CHUNK_FLAT, EMB), jnp.float32),
            pltpu.VMEM((CHUNK_FLAT, EMB), jnp.float32),
            pltpu.VMEM((BPW * EMB,), jnp.float32),
            pltpu.SemaphoreType.DMA,
            pltpu.SemaphoreType.DMA,
        ],
        compiler_params=pltpu.CompilerParams(use_tc_tiling_on_sc=False),
    )(x2d, emb)


VB = 32768
NVB = (VOCAB + VB - 1) // VB
VOCAB_P = NVB * VB
QSH = (VB // 4).bit_length() - 1


def _prep_body(et_ref, o_ref):
    q = VB // 4
    row = lax.broadcasted_iota(jnp.int32, (EMB, 128), 0)
    col = lax.broadcasted_iota(jnp.int32, (EMB, 128), 1)
    acc = jnp.zeros((q, 128), jnp.float32)
    for a in range(4):
        ea = jnp.where(col == 32 * a + row, 1.0, 0.0).astype(jnp.float32)
        acc = acc + lax.dot_general(
            et_ref[:, pl.ds(a * q, q)], ea, (((0,), (0,)), ((), ())),
            preferred_element_type=jnp.float32)
    o_ref[...] = acc


def _prep(embT):
    return pl.pallas_call(
        _prep_body,
        grid=(NVB,),
        in_specs=[pl.BlockSpec((EMB, VB), lambda i: (0, i))],
        out_specs=pl.BlockSpec((VB // 4, 128), lambda i: (i, 0)),
        out_shape=jax.ShapeDtypeStruct((VOCAB_P * EMB // 128, 128),
                                       jnp.float32),
    )(embT)


def _mlp_body(h_ref, w1_ref, b1_ref, w2_ref, b2_ref, o_ref):
    h = h_ref[...] * jnp.float32(1.0 / L)
    z = jnp.dot(h, w1_ref[...], preferred_element_type=jnp.float32)
    z = jnp.maximum(z + b1_ref[...], 0.0)
    o_ref[...] = jnp.dot(z, w2_ref[...],
                         preferred_element_type=jnp.float32) + b2_ref[...]


def _mlp(pooled, w1t, b1r, w2p, b2p):
    BM = 1024
    grid = (B // BM,)
    return pl.pallas_call(
        _mlp_body,
        grid=grid,
        in_specs=[
            pl.BlockSpec((BM, EMB), lambda i: (i, 0)),
            pl.BlockSpec((EMB, HID), lambda i: (0, 0)),
            pl.BlockSpec((1, HID), lambda i: (0, 0)),
            pl.BlockSpec((HID, 128), lambda i: (0, 0)),
            pl.BlockSpec((1, 128), lambda i: (0, 0)),
        ],
        out_specs=pl.BlockSpec((BM, 128), lambda i: (i, 0)),
        out_shape=jax.ShapeDtypeStruct((B, 128), jnp.float32),
    )(pooled, w1t, b1r, w2p, b2p)


def kernel(x, emb, W1, b1, W2, b2):
    xi = x.astype(jnp.int32)
    x2d = (xi & -VB) + ((xi & (VB // 4 - 1)) << 2) + ((xi >> QSH) & 3)
    emb_in = _prep(emb.T).reshape(VOCAB_P, EMB)
    pooled = _sc_pool(x2d, emb_in).reshape(B, EMB)
    w1t = W1.T
    w2p = jnp.pad(W2.T, ((0, 0), (0, 128 - NCLS)))
    b2p = jnp.pad(b2, (0, 128 - NCLS)).reshape(1, 128)
    out = _mlp(pooled, w1t, b1.reshape(1, HID), w2p, b2p)
    return out[:, :NCLS]


# speedup vs baseline: 3.0849x; 1.0010x over previous
import jax
import jax.numpy as jnp
from jax import lax
from jax.experimental import pallas as pl
from jax.experimental.pallas import tpu as pltpu
from jax.experimental.pallas import tpu_sc as plsc

VOCAB = 1000000
EMB = 32
HID = 128
NCLS = 3
B = 16384
L = 200

NC = 2
NS = 16
NW = NC * NS
BPW = B // NW
IDX_COLS = 100
ROWS_PER_BR = L // IDX_COLS
BR_PER_CHUNK = 8
CHUNK_IDX_ROWS = BR_PER_CHUNK * ROWS_PER_BR
CHUNK_FLAT = BR_PER_CHUNK * L
IDX_ROWS_PER_W = BPW * ROWS_PER_BR
NCHUNKS = BPW // BR_PER_CHUNK
UNROLL = 8
NACC = 4


def _pool_body(x_hbm, emb_hbm, out_hbm, idx0, idx1, rows0, rows1, acc_v,
               sem0, sem1):
    cid = lax.axis_index("c")
    sid = lax.axis_index("s")
    wid = sid * NC + cid
    row0 = wid * BPW

    zero16 = jnp.zeros((16,), jnp.float32)

    def chunk_copies(idxbuf, rowsbuf, sem):
        for j in range(CHUNK_IDX_ROWS):
            br, half = j // 2, j % 2
            off, sz = (0, 104) if half == 0 else (104, 96)
            yield (emb_hbm.at[idxbuf.at[br, pl.ds(off, sz)]],
                   rowsbuf.at[pl.ds(br * L + off, sz)], sem)

    def fire(c, idxbuf, rowsbuf, sem):
        pltpu.sync_copy(
            x_hbm.at[pl.ds(row0 + c * BR_PER_CHUNK, BR_PER_CHUNK)],
            idxbuf)
        for src, dst, s in chunk_copies(idxbuf, rowsbuf, sem):
            pltpu.async_copy(src, dst, s)

    def drain(idxbuf, rowsbuf, sem):
        for src, dst, s in chunk_copies(idxbuf, rowsbuf, sem):
            pltpu.make_async_copy(src, dst, s).wait()

    def accumulate(c, rowsbuf):
        for b in range(BR_PER_CHUNK):
            lbase = (c * BR_PER_CHUNK + b) * EMB

            def inner(j, accs, _b=b):
                accs = list(accs)
                rr0 = _b * L + j * UNROLL
                for k in range(UNROLL):
                    a = k % NACC
                    accs[2 * a] = accs[2 * a] + rowsbuf[rr0 + k, pl.ds(0, 16)]
                    accs[2 * a + 1] = (accs[2 * a + 1]
                                       + rowsbuf[rr0 + k, pl.ds(16, 16)])
                return tuple(accs)

            accs = lax.fori_loop(0, L // UNROLL, inner,
                                 (zero16,) * (2 * NACC))
            a0 = accs[0]
            a1 = accs[1]
            for a in range(1, NACC):
                a0 = a0 + accs[2 * a]
                a1 = a1 + accs[2 * a + 1]
            acc_v[pl.ds(lbase, 16)] = a0
            acc_v[pl.ds(lbase + 16, 16)] = a1

    fire(0, idx0, rows0, sem0)

    def outer(c2, carry):
        c = c2 * 2
        fire(c + 1, idx1, rows1, sem1)
        drain(idx0, rows0, sem0)
        accumulate(c, rows0)

        @pl.when(c2 < NCHUNKS // 2 - 1)
        def _():
            fire(c + 2, idx0, rows0, sem0)

        drain(idx1, rows1, sem1)
        accumulate(c + 1, rows1)
        return carry

    lax.fori_loop(0, NCHUNKS // 2, outer, 0)
    pltpu.sync_copy(acc_v, out_hbm.at[pl.ds(wid * BPW * EMB, BPW * EMB)])


@jax.jit
def _sc_pool(x2d, emb):
    mesh = plsc.VectorSubcoreMesh(core_axis_name="c", subcore_axis_name="s")
    return pl.kernel(
        _pool_body,
        out_type=jax.ShapeDtypeStruct((B * EMB,), jnp.float32),
        mesh=mesh,
        scratch_types=[
            pltpu.VMEM((BR_PER_CHUNK, L), jnp.int32),
            pltpu.VMEM((BR_PER_CHUNK, L), jnp.int32),
            pltpu.VMEM((CHUNK_FLAT, EMB), jnp.float32),
            pltpu.VMEM((CHUNK_FLAT, EMB), jnp.float32),
            pltpu.VMEM((BPW * EMB,), jnp.float32),
            pltpu.SemaphoreType.DMA,
            pltpu.SemaphoreType.DMA,
        ],
        compiler_params=pltpu.CompilerParams(use_tc_tiling_on_sc=False),
    )(x2d, emb)


VB = 32768
NVB = (VOCAB + VB - 1) // VB
VOCAB_P = NVB * VB
QSH = (VB // 4).bit_length() - 1


def _prep_body(et_ref, o_ref):
    q = VB // 4
    row = lax.broadcasted_iota(jnp.int32, (EMB, 128), 0)
    col = lax.broadcasted_iota(jnp.int32, (EMB, 128), 1)
    acc = jnp.zeros((q, 128), jnp.float32)
    for a in range(4):
        ea = jnp.where(col == 32 * a + row, 1.0, 0.0).astype(jnp.float32)
        acc = acc + lax.dot_general(
            et_ref[:, pl.ds(a * q, q)], ea, (((0,), (0,)), ((), ())),
            preferred_element_type=jnp.float32)
    o_ref[...] = acc


def _prep(embT):
    return pl.pallas_call(
        _prep_body,
        grid=(NVB,),
        in_specs=[pl.BlockSpec((EMB, VB), lambda i: (0, i))],
        out_specs=pl.BlockSpec((VB // 4, 128), lambda i: (i, 0)),
        out_shape=jax.ShapeDtypeStruct((VOCAB_P * EMB // 128, 128),
                                       jnp.float32),
    )(embT)


def _mlp_body(h_ref, w1_ref, b1_ref, w2_ref, b2_ref, o_ref):
    h = h_ref[...] * jnp.float32(1.0 / L)
    z = jnp.dot(h, w1_ref[...], preferred_element_type=jnp.float32)
    z = jnp.maximum(z + b1_ref[...], 0.0)
    o_ref[...] = jnp.dot(z, w2_ref[...],
                         preferred_element_type=jnp.float32) + b2_ref[...]


def _mlp(pooled, w1t, b1r, w2p, b2p):
    BM = 1024
    grid = (B // BM,)
    return pl.pallas_call(
        _mlp_body,
        grid=grid,
        in_specs=[
            pl.BlockSpec((BM, EMB), lambda i: (i, 0)),
            pl.BlockSpec((EMB, HID), lambda i: (0, 0)),
            pl.BlockSpec((1, HID), lambda i: (0, 0)),
            pl.BlockSpec((HID, NCLS), lambda i: (0, 0)),
            pl.BlockSpec((1, NCLS), lambda i: (0, 0)),
        ],
        out_specs=pl.BlockSpec((BM, NCLS), lambda i: (i, 0)),
        out_shape=jax.ShapeDtypeStruct((B, NCLS), jnp.float32),
    )(pooled, w1t, b1r, w2p, b2p)


def kernel(x, emb, W1, b1, W2, b2):
    xi = x.astype(jnp.int32)
    x2d = (xi & -VB) + ((xi & (VB // 4 - 1)) << 2) + ((xi >> QSH) & 3)
    emb_in = _prep(emb.T).reshape(VOCAB_P, EMB)
    pooled = _sc_pool(x2d, emb_in).reshape(B, EMB)
    w1t = W1.T
    return _mlp(pooled, w1t, b1.reshape(1, HID), W2.T, b2.reshape(1, NCLS))
